# baseline (speedup 1.0000x reference)
import functools

import jax
import jax.numpy as jnp
from jax import lax
from jax.experimental import pallas as pl
from jax.experimental.pallas import tpu as pltpu
from jax.experimental.pallas import tpu_sc as plsc

ENT_ROWS = 1000000
REL_ROWS = 1000
EMB_DIM = 64
BATCH = 16384
NUM_CORES = 2
NUM_SUBCORES = 16
NUM_WORKERS = NUM_CORES * NUM_SUBCORES
BPW = BATCH // NUM_WORKERS
CHUNK = 16
NCHUNKS = BPW // CHUNK
NPAIRS = NCHUNKS // 2

_mesh = plsc.VectorSubcoreMesh(core_axis_name="c", subcore_axis_name="s")


@functools.partial(
    pl.kernel,
    mesh=_mesh,
    out_type=jax.ShapeDtypeStruct((EMB_DIM, BATCH), jnp.float32),
    compiler_params=pltpu.CompilerParams(use_tc_tiling_on_sc=True,
                                         needs_layout_passes=False),
    scratch_types=[
        pltpu.VMEM((BPW,), jnp.int32),
        pltpu.VMEM((BPW,), jnp.int32),
        pltpu.VMEM((BPW,), jnp.int32),
        pltpu.VMEM((CHUNK,), jnp.int32),
        pltpu.VMEM((CHUNK,), jnp.int32),
        pltpu.VMEM((CHUNK, 8, EMB_DIM), jnp.float32),
        pltpu.VMEM((CHUNK, 8, EMB_DIM), jnp.float32),
        pltpu.VMEM((CHUNK, 128), jnp.float32),
        pltpu.VMEM((CHUNK, 8, EMB_DIM), jnp.float32),
        pltpu.VMEM((CHUNK, 8, EMB_DIM), jnp.float32),
        pltpu.VMEM((CHUNK, 128), jnp.float32),
        pltpu.VMEM((EMB_DIM, BPW), jnp.float32),
        pltpu.SemaphoreType.DMA,
        pltpu.SemaphoreType.DMA,
    ],
)
def _transe_sc(ent_hbm, rel_hbm, head_hbm, ridx_hbm, tail_hbm, out_hbm,
               idxh, idxr, idxt, girA, girB, hA, tA, rA, hB, tB, rB,
               outb, semA, semB):
    wid = lax.axis_index("s") * NUM_CORES + lax.axis_index("c")
    base = wid * BPW

    pltpu.sync_copy(head_hbm.at[pl.ds(base, BPW)], idxh)
    pltpu.sync_copy(ridx_hbm.at[pl.ds(base, BPW)], idxr)
    pltpu.sync_copy(tail_hbm.at[pl.ds(base, BPW)], idxt)

    iota = lax.iota(jnp.int32, 16)

    def enqueue(cidx, hb, tb, rb, gir, sem):
        sl = pl.ds(cidx * CHUNK, CHUNK)
        th = idxh[sl] >> 3
        tt = idxt[sl] >> 3
        gir[...] = idxr[sl] >> 1
        pltpu.async_copy(rel_hbm.at[gir], rb, sem)
        for j in range(CHUNK):
            pltpu.async_copy(ent_hbm.at[th[j], :, :], hb.at[j], sem)
            pltpu.async_copy(ent_hbm.at[tt[j], :, :], tb.at[j], sem)

    def drain(hb, tb, rb, sem):
        dummy = ent_hbm.at[pl.ds(0, CHUNK)]
        pltpu.make_async_copy(dummy, hb, sem).wait()
        pltpu.make_async_copy(dummy, tb, sem).wait()
        pltpu.make_async_copy(rel_hbm.at[pl.ds(0, CHUNK)], rb, sem).wait()

    def extract_write(cidx, hb, tb, rb):
        sl = pl.ds(cidx * CHUNK, CHUNK)
        rows_h = idxh[sl] & 7
        rows_t = idxt[sl] & 7
        o_r = (idxr[sl] & 1) << 6
        obase = cidx * CHUNK

        def extract(d, c):
            dv = jnp.broadcast_to(d, (16,))
            hv = plsc.load_gather(hb, [iota, rows_h, dv])
            tv = plsc.load_gather(tb, [iota, rows_t, dv])
            rv = plsc.load_gather(rb, [iota, o_r + d])
            outb[d, pl.ds(obase, CHUNK)] = hv + rv - tv
            return c

        lax.fori_loop(0, EMB_DIM, extract, 0)

    enqueue(0, hA, tA, rA, girA, semA)

    def pairbody(p, carry):
        c0 = p * 2
        enqueue(c0 + 1, hB, tB, rB, girB, semB)
        drain(hA, tA, rA, semA)
        extract_write(c0, hA, tA, rA)

        @pl.when(p < NPAIRS - 1)
        def _():
            enqueue(c0 + 2, hA, tA, rA, girA, semA)

        drain(hB, tB, rB, semB)
        extract_write(c0 + 1, hB, tB, rB)
        return carry

    lax.fori_loop(0, NPAIRS, pairbody, 0)

    pltpu.sync_copy(outb, out_hbm.at[:, pl.ds(base, BPW)])


def kernel(head, relation, tail, ent_emb, rel_emb):
    out_t = _transe_sc(
        ent_emb.reshape(ENT_ROWS // 8, 8, EMB_DIM),
        rel_emb.reshape(REL_ROWS // 2, 128),
        head.reshape(BATCH),
        relation.reshape(BATCH),
        tail.reshape(BATCH),
    )
    return out_t.T


# speedup vs baseline: 1.0072x; 1.0072x over previous
import functools

import jax
import jax.numpy as jnp
from jax import lax
from jax.experimental import pallas as pl
from jax.experimental.pallas import tpu as pltpu
from jax.experimental.pallas import tpu_sc as plsc

ENT_ROWS = 1000000
REL_ROWS = 1000
EMB_DIM = 64
BATCH = 16384
NUM_CORES = 2
NUM_SUBCORES = 16
NUM_WORKERS = NUM_CORES * NUM_SUBCORES
BPW = BATCH // NUM_WORKERS
CHUNK = 16
NCHUNKS = BPW // CHUNK
NPAIRS = NCHUNKS // 2

_mesh = plsc.VectorSubcoreMesh(core_axis_name="c", subcore_axis_name="s")


@functools.partial(
    pl.kernel,
    mesh=_mesh,
    out_type=jax.ShapeDtypeStruct((BATCH, EMB_DIM), jnp.float32),
    compiler_params=pltpu.CompilerParams(use_tc_tiling_on_sc=True,
                                         needs_layout_passes=False),
    scratch_types=[
        pltpu.VMEM((BPW,), jnp.int32),
        pltpu.VMEM((BPW,), jnp.int32),
        pltpu.VMEM((BPW,), jnp.int32),
        pltpu.VMEM((CHUNK,), jnp.int32),
        pltpu.VMEM((CHUNK,), jnp.int32),
        pltpu.VMEM((CHUNK * 8, EMB_DIM), jnp.float32),
        pltpu.VMEM((CHUNK * 8, EMB_DIM), jnp.float32),
        pltpu.VMEM((CHUNK, 128), jnp.float32),
        pltpu.VMEM((CHUNK * 8, EMB_DIM), jnp.float32),
        pltpu.VMEM((CHUNK * 8, EMB_DIM), jnp.float32),
        pltpu.VMEM((CHUNK, 128), jnp.float32),
        pltpu.VMEM((CHUNK, EMB_DIM), jnp.float32),
        pltpu.SemaphoreType.DMA,
        pltpu.SemaphoreType.DMA,
    ],
)
def _transe_sc(ent_hbm, rel_hbm, head_hbm, ridx_hbm, tail_hbm, out_hbm,
               idxh, idxr, idxt, girA, girB, hA, tA, rA, hB, tB, rB,
               outb, semA, semB):
    wid = lax.axis_index("s") * NUM_CORES + lax.axis_index("c")
    base = wid * BPW

    pltpu.sync_copy(head_hbm.at[pl.ds(base, BPW)], idxh)
    pltpu.sync_copy(ridx_hbm.at[pl.ds(base, BPW)], idxr)
    pltpu.sync_copy(tail_hbm.at[pl.ds(base, BPW)], idxt)

    iota = lax.iota(jnp.int32, 16)

    def enqueue(cidx, hb, tb, rb, gir, sem):
        sl = pl.ds(cidx * CHUNK, CHUNK)
        th = idxh[sl] >> 3
        tt = idxt[sl] >> 3
        gir[...] = idxr[sl] >> 1
        pltpu.async_copy(rel_hbm.at[gir], rb, sem)
        for j in range(CHUNK):
            dsl = pl.ds(j * 8, 8)
            pltpu.async_copy(ent_hbm.at[th[j], :, :], hb.at[dsl, :], sem)
            pltpu.async_copy(ent_hbm.at[tt[j], :, :], tb.at[dsl, :], sem)

    def drain(hb, tb, rb, sem):
        dummy = out_hbm.at[pl.ds(0, CHUNK * 8), :]
        pltpu.make_async_copy(dummy, hb, sem).wait()
        pltpu.make_async_copy(dummy, tb, sem).wait()
        pltpu.make_async_copy(rel_hbm.at[pl.ds(0, CHUNK)], rb, sem).wait()

    def extract_write(cidx, hb, tb, rb):
        sl = pl.ds(cidx * CHUNK, CHUNK)
        s_h = idxh[sl] & 7
        s_t = idxt[sl] & 7
        o_r = (idxr[sl] & 1) << 6

        def extract(d, c):
            cols = d * 16 + iota
            for j in range(CHUNK):
                hv = plsc.load_gather(
                    hb, [jnp.broadcast_to(j * 8 + s_h[j], (16,)), cols])
                tv = plsc.load_gather(
                    tb, [jnp.broadcast_to(j * 8 + s_t[j], (16,)), cols])
                rv = rb[j, pl.ds(o_r[j] + d * 16, 16)]
                outb[j, pl.ds(d * 16, 16)] = hv + rv - tv
            return c

        lax.fori_loop(0, EMB_DIM // 16, extract, 0)
        pltpu.sync_copy(outb, out_hbm.at[pl.ds(base + cidx * CHUNK, CHUNK)])

    enqueue(0, hA, tA, rA, girA, semA)

    def pairbody(p, carry):
        c0 = p * 2
        enqueue(c0 + 1, hB, tB, rB, girB, semB)
        drain(hA, tA, rA, semA)
        extract_write(c0, hA, tA, rA)

        @pl.when(p < NPAIRS - 1)
        def _():
            enqueue(c0 + 2, hA, tA, rA, girA, semA)

        drain(hB, tB, rB, semB)
        extract_write(c0 + 1, hB, tB, rB)
        return carry

    lax.fori_loop(0, NPAIRS, pairbody, 0)


def kernel(head, relation, tail, ent_emb, rel_emb):
    return _transe_sc(
        ent_emb.reshape(ENT_ROWS // 8, 8, EMB_DIM),
        rel_emb.reshape(REL_ROWS // 2, 128),
        head.reshape(BATCH),
        relation.reshape(BATCH),
        tail.reshape(BATCH),
    )


# speedup vs baseline: 1.0211x; 1.0138x over previous
import functools

import jax
import jax.numpy as jnp
from jax import lax
from jax.experimental import pallas as pl
from jax.experimental.pallas import tpu as pltpu
from jax.experimental.pallas import tpu_sc as plsc

ENT_ROWS = 1000000
REL_ROWS = 1000
EMB_DIM = 64
BATCH = 16384
NUM_CORES = 2
NUM_SUBCORES = 16
NUM_WORKERS = NUM_CORES * NUM_SUBCORES
BPW = BATCH // NUM_WORKERS
CHUNK = 16
NCHUNKS = BPW // CHUNK
NTRIPLES = (NCHUNKS - 2) // 3

_mesh = plsc.VectorSubcoreMesh(core_axis_name="c", subcore_axis_name="s")


@functools.partial(
    pl.kernel,
    mesh=_mesh,
    out_type=jax.ShapeDtypeStruct((BATCH, EMB_DIM), jnp.float32),
    compiler_params=pltpu.CompilerParams(use_tc_tiling_on_sc=True,
                                         needs_layout_passes=False),
    scratch_types=[
        pltpu.VMEM((BPW,), jnp.int32),
        pltpu.VMEM((BPW,), jnp.int32),
        pltpu.VMEM((BPW,), jnp.int32),
        pltpu.VMEM((CHUNK,), jnp.int32),
        pltpu.VMEM((CHUNK,), jnp.int32),
        pltpu.VMEM((CHUNK,), jnp.int32),
        pltpu.VMEM((CHUNK * 8, EMB_DIM), jnp.float32),
        pltpu.VMEM((CHUNK * 8, EMB_DIM), jnp.float32),
        pltpu.VMEM((CHUNK, 128), jnp.float32),
        pltpu.VMEM((CHUNK * 8, EMB_DIM), jnp.float32),
        pltpu.VMEM((CHUNK * 8, EMB_DIM), jnp.float32),
        pltpu.VMEM((CHUNK, 128), jnp.float32),
        pltpu.VMEM((CHUNK * 8, EMB_DIM), jnp.float32),
        pltpu.VMEM((CHUNK * 8, EMB_DIM), jnp.float32),
        pltpu.VMEM((CHUNK, 128), jnp.float32),
        pltpu.VMEM((CHUNK, EMB_DIM), jnp.float32),
        pltpu.SemaphoreType.DMA,
        pltpu.SemaphoreType.DMA,
        pltpu.SemaphoreType.DMA,
    ],
)
def _transe_sc(ent_hbm, rel_hbm, head_hbm, ridx_hbm, tail_hbm, out_hbm,
               idxh, idxr, idxt, girA, girB, girC, hA, tA, rA, hB, tB, rB,
               hC, tC, rC, outb, semA, semB, semC):
    wid = lax.axis_index("s") * NUM_CORES + lax.axis_index("c")
    base = wid * BPW

    pltpu.sync_copy(head_hbm.at[pl.ds(base, BPW)], idxh)
    pltpu.sync_copy(ridx_hbm.at[pl.ds(base, BPW)], idxr)
    pltpu.sync_copy(tail_hbm.at[pl.ds(base, BPW)], idxt)

    iota = lax.iota(jnp.int32, 16)

    def enqueue(cidx, hb, tb, rb, gir, sem):
        sl = pl.ds(cidx * CHUNK, CHUNK)
        th = idxh[sl] >> 3
        tt = idxt[sl] >> 3
        gir[...] = idxr[sl] >> 1
        pltpu.async_copy(rel_hbm.at[gir], rb, sem)
        for j in range(CHUNK):
            dsl = pl.ds(j * 8, 8)
            pltpu.async_copy(ent_hbm.at[th[j], :, :], hb.at[dsl, :], sem)
            pltpu.async_copy(ent_hbm.at[tt[j], :, :], tb.at[dsl, :], sem)

    def drain(hb, tb, rb, sem):
        dummy = out_hbm.at[pl.ds(0, CHUNK * 8), :]
        pltpu.make_async_copy(dummy, hb, sem).wait()
        pltpu.make_async_copy(dummy, tb, sem).wait()
        pltpu.make_async_copy(rel_hbm.at[pl.ds(0, CHUNK)], rb, sem).wait()

    def extract_write(cidx, hb, tb, rb):
        sl = pl.ds(cidx * CHUNK, CHUNK)
        s_h = idxh[sl] & 7
        s_t = idxt[sl] & 7
        o_r = (idxr[sl] & 1) << 6

        def extract(d, c):
            cols = d * 16 + iota
            for j in range(CHUNK):
                hv = plsc.load_gather(
                    hb, [jnp.broadcast_to(j * 8 + s_h[j], (16,)), cols])
                tv = plsc.load_gather(
                    tb, [jnp.broadcast_to(j * 8 + s_t[j], (16,)), cols])
                rv = rb[j, pl.ds(o_r[j] + d * 16, 16)]
                outb[j, pl.ds(d * 16, 16)] = hv + rv - tv
            return c

        lax.fori_loop(0, EMB_DIM // 16, extract, 0)
        pltpu.sync_copy(outb, out_hbm.at[pl.ds(base + cidx * CHUNK, CHUNK)])

    enqueue(0, hA, tA, rA, girA, semA)
    enqueue(1, hB, tB, rB, girB, semB)

    def triplebody(q, carry):
        c0 = q * 3
        enqueue(c0 + 2, hC, tC, rC, girC, semC)
        drain(hA, tA, rA, semA)
        extract_write(c0, hA, tA, rA)
        enqueue(c0 + 3, hA, tA, rA, girA, semA)
        drain(hB, tB, rB, semB)
        extract_write(c0 + 1, hB, tB, rB)
        enqueue(c0 + 4, hB, tB, rB, girB, semB)
        drain(hC, tC, rC, semC)
        extract_write(c0 + 2, hC, tC, rC)
        return carry

    lax.fori_loop(0, NTRIPLES, triplebody, 0)

    drain(hA, tA, rA, semA)
    extract_write(NCHUNKS - 2, hA, tA, rA)
    drain(hB, tB, rB, semB)
    extract_write(NCHUNKS - 1, hB, tB, rB)


def kernel(head, relation, tail, ent_emb, rel_emb):
    return _transe_sc(
        ent_emb.reshape(ENT_ROWS // 8, 8, EMB_DIM),
        rel_emb.reshape(REL_ROWS // 2, 128),
        head.reshape(BATCH),
        relation.reshape(BATCH),
        tail.reshape(BATCH),
    )
